# Initial kernel scaffold; baseline (speedup 1.0000x reference)
import jax
import jax.numpy as jnp
from jax.experimental import pallas as pl


def kernel(x_account, x_merchant, edge_index_pays, edge_index_receives, edge_index_transfers, Wp_account, bp_account, Wp_merchant, bp_merchant, Wrel0, Wroot0, b0, gamma0, beta0, Wrel1, Wroot1, b1, gamma1, beta1, Wrel2, Wroot2, b2, gamma2, beta2, Wc1, bc1, Wc2, bc2):
    raise NotImplementedError("write your pallas kernel here")


# speedup vs baseline: 5.3065x; 5.3065x over previous
import functools
import jax
import jax.numpy as jnp
from jax import lax
from jax.experimental import pallas as pl
from jax.experimental.pallas import tpu as pltpu
from jax.experimental.pallas import tpu_sc as plsc

N_ACC = 50000
N_MER = 10000
N = N_ACC + N_MER

NTILES = 16
CHUNK = 128
ACC_ROWS = 10112
GR = 10000
ZROWS = ACC_ROWS // NTILES
VALID = 10000
PER = 624

E0P = 251904
E2P = 100352
T0 = E0P // NTILES
T2 = E2P // NTILES

def _sc_writeout(acc, out_hbm, s, ob):
    pltpu.sync_copy(acc.at[pl.ds(s * PER, PER)], out_hbm.at[pl.ds(ob + s * PER, PER)])
    rem = VALID - PER * NTILES
    @pl.when(s == 0)
    def _():
        pltpu.sync_copy(acc.at[pl.ds(PER * NTILES, rem)],
                        out_hbm.at[pl.ds(ob + PER * NTILES, rem)])


def _run_slot(srcR, dstR, ept, nchunk, ob, x_hbm, out_hbm, zeros_hbm,
              idx_v, dst_v, rows_v, acc, sem, s):
    pltpu.sync_copy(zeros_hbm, acc.at[pl.ds(s * ZROWS, ZROWS)])
    plsc.subcore_barrier()
    base_e = s * ept

    def body(g, carry):
        off = base_e + g * CHUNK
        pltpu.sync_copy(srcR.at[pl.ds(off, CHUNK)], idx_v)
        pltpu.sync_copy(dstR.at[pl.ds(off, CHUNK)], dst_v)
        pltpu.async_copy(x_hbm.at[idx_v], rows_v, sem).wait()
        pltpu.sync_copy(rows_v, acc.at[dst_v], add=True)
        return carry

    lax.fori_loop(0, nchunk, body, 0)
    plsc.subcore_barrier()
    _sc_writeout(acc, out_hbm, s, ob)
    plsc.subcore_barrier()


@functools.lru_cache(maxsize=None)
def _sc_feature_pass_k():
    mesh = plsc.VectorSubcoreMesh(core_axis_name="c", subcore_axis_name="s")

    @functools.partial(
        pl.kernel, mesh=mesh,
        out_type=jax.ShapeDtypeStruct((70000, 128), jnp.float32),
        scratch_types=[
            pltpu.VMEM((CHUNK,), jnp.int32),
            pltpu.VMEM((CHUNK,), jnp.int32),
            pltpu.VMEM((CHUNK, 128), jnp.float32),
            pltpu.VMEM_SHARED((ACC_ROWS, 128), jnp.float32),
            pltpu.SemaphoreType.DMA,
        ])
    def _sc_feature_pass(x_hbm, src_p, dst_p, src_r, dst_r, src_t,
                         dst_c0, dst_c1, dst_c2, dst_c3, dst_c4, zeros_hbm,
                         out_hbm, idx_v, dst_v, rows_v, acc, sem):
        c = lax.axis_index("c")
        s = lax.axis_index("s")
        core_slots = (
            ((src_p, dst_p, T0, T0 // CHUNK, 0),
             (src_t, dst_c0, T2, T2 // CHUNK, 20000),
             (src_t, dst_c1, T2, T2 // CHUNK, 30000)),
            ((src_r, dst_r, T0, T0 // CHUNK, 10000),
             (src_t, dst_c2, T2, T2 // CHUNK, 40000),
             (src_t, dst_c3, T2, T2 // CHUNK, 50000),
             (src_t, dst_c4, T2, T2 // CHUNK, 60000)),
        )
        for core_id in (0, 1):
            @pl.when(c == core_id)
            def _():
                for (srcR, dstR, ept, nchunk, ob) in core_slots[core_id]:
                    _run_slot(srcR, dstR, ept, nchunk, ob, x_hbm, out_hbm,
                              zeros_hbm, idx_v, dst_v, rows_v, acc, sem, s)

    return _sc_feature_pass


def _run_cslot(dstR, ept, nchunk, ob, out_hbm, zerosc_hbm,
               dst_v, ones_v, acc, sem, s):
    pltpu.sync_copy(zerosc_hbm, acc.at[pl.ds(s * ZROWS, ZROWS)])
    plsc.subcore_barrier()
    base_e = s * ept

    def body(g, carry):
        off = base_e + g * CHUNK
        pltpu.sync_copy(dstR.at[pl.ds(off, CHUNK)], dst_v)
        pltpu.sync_copy(ones_v, acc.at[dst_v], add=True)
        return carry

    lax.fori_loop(0, nchunk, body, 0)
    plsc.subcore_barrier()
    _sc_writeout(acc, out_hbm, s, ob)
    plsc.subcore_barrier()


@functools.lru_cache(maxsize=None)
def _sc_counts_pass_k():
    mesh = plsc.VectorSubcoreMesh(core_axis_name="c", subcore_axis_name="s")

    @functools.partial(
        pl.kernel, mesh=mesh,
        out_type=jax.ShapeDtypeStruct((70000, 128), jnp.float32),
        scratch_types=[
            pltpu.VMEM((CHUNK,), jnp.int32),
            pltpu.VMEM((CHUNK, 128), jnp.float32),
            pltpu.VMEM_SHARED((ACC_ROWS, 128), jnp.float32),
            pltpu.SemaphoreType.DMA,
        ])
    def _sc_counts_pass(dst_p, dst_r, dst_c0, dst_c1, dst_c2, dst_c3, dst_c4,
                        zerosc_hbm, ones_hbm,
                        out_hbm, dst_v, ones_v, acc, sem):
        c = lax.axis_index("c")
        s = lax.axis_index("s")
        pltpu.sync_copy(ones_hbm, ones_v)
        core_slots = (
            ((dst_p, T0, T0 // CHUNK, 0),
             (dst_c0, T2, T2 // CHUNK, 20000),
             (dst_c1, T2, T2 // CHUNK, 30000)),
            ((dst_r, T0, T0 // CHUNK, 10000),
             (dst_c2, T2, T2 // CHUNK, 40000),
             (dst_c3, T2, T2 // CHUNK, 50000),
             (dst_c4, T2, T2 // CHUNK, 60000)),
        )
        for core_id in (0, 1):
            @pl.when(c == core_id)
            def _():
                for (dstR, ept, nchunk, ob) in core_slots[core_id]:
                    _run_cslot(dstR, ept, nchunk, ob, out_hbm, zerosc_hbm,
                               dst_v, ones_v, acc, sem, s)

    return _sc_counts_pass


BR = 2000


def _proj_body(x_ref, w_ref, b_ref, o_ref):
    o_ref[...] = jnp.dot(x_ref[...], w_ref[...],
                         preferred_element_type=jnp.float32) + b_ref[...]


def _tc_proj(x, w, b):
    n = x.shape[0]
    return pl.pallas_call(
        _proj_body,
        grid=(n // BR,),
        in_specs=[pl.BlockSpec((BR, 128), lambda i: (i, 0)),
                  pl.BlockSpec((128, 128), lambda i: (0, 0)),
                  pl.BlockSpec((1, 128), lambda i: (0, 0))],
        out_specs=pl.BlockSpec((BR, 128), lambda i: (i, 0)),
        out_shape=jax.ShapeDtypeStruct((n, 128), jnp.float32),
    )(x, w, b.reshape(1, 128))


_BN_SCALE = 1.0 / (1.0 + 1e-5) ** 0.5


def _mean(s_ref, c_ref):
    inv = 1.0 / jnp.maximum(c_ref[...][:, 0:1], 1.0)
    return s_ref[...] * inv


def _combine1_body(x_ref, wroot_ref, s_ref, c_ref, w_ref, b_ref, g_ref, be_ref,
                   o_ref, *, relu):
    acc = jnp.dot(x_ref[...], wroot_ref[...], preferred_element_type=jnp.float32)
    acc += jnp.dot(_mean(s_ref, c_ref), w_ref[...], preferred_element_type=jnp.float32)
    acc += b_ref[...]
    acc = g_ref[...] * acc * _BN_SCALE + be_ref[...]
    o_ref[...] = jnp.maximum(acc, 0.0) if relu else acc


def _combine2_body(x_ref, wroot_ref, s1_ref, c1_ref, w1_ref, s2_ref, c2_ref, w2_ref,
                   b_ref, g_ref, be_ref, o_ref, *, relu):
    acc = jnp.dot(x_ref[...], wroot_ref[...], preferred_element_type=jnp.float32)
    acc += jnp.dot(_mean(s1_ref, c1_ref), w1_ref[...], preferred_element_type=jnp.float32)
    acc += jnp.dot(_mean(s2_ref, c2_ref), w2_ref[...], preferred_element_type=jnp.float32)
    acc += b_ref[...]
    acc = g_ref[...] * acc * _BN_SCALE + be_ref[...]
    o_ref[...] = jnp.maximum(acc, 0.0) if relu else acc


def _row_spec(off, w):
    o = off // BR
    return pl.BlockSpec((BR, w), lambda i, o=o: (i + o, 0))


def _full_spec(r, cdim):
    return pl.BlockSpec((r, cdim), lambda i: (0, 0))


def _combine1(nrows, x, xoff, S, soff, cnt, wroot, w, b, g, be, relu):
    dout = w.shape[1]
    body = functools.partial(_combine1_body, relu=relu)
    return pl.pallas_call(
        body,
        grid=(nrows // BR,),
        in_specs=[_row_spec(xoff, 128), _full_spec(128, dout),
                  _row_spec(soff, 128), _row_spec(soff, 128), _full_spec(128, dout),
                  _full_spec(1, dout), _full_spec(1, dout), _full_spec(1, dout)],
        out_specs=pl.BlockSpec((BR, dout), lambda i: (i, 0)),
        out_shape=jax.ShapeDtypeStruct((nrows, dout), jnp.float32),
    )(x, wroot, S, cnt, w, b.reshape(1, dout), g.reshape(1, dout), be.reshape(1, dout))


def _combine2(nrows, x, xoff, S, s1off, s2off, cnt, wroot, w1, w2, b, g, be, relu):
    dout = w1.shape[1]
    body = functools.partial(_combine2_body, relu=relu)
    return pl.pallas_call(
        body,
        grid=(nrows // BR,),
        in_specs=[_row_spec(xoff, 128), _full_spec(128, dout),
                  _row_spec(s1off, 128), _row_spec(s1off, 128), _full_spec(128, dout),
                  _row_spec(s2off, 128), _row_spec(s2off, 128), _full_spec(128, dout),
                  _full_spec(1, dout), _full_spec(1, dout), _full_spec(1, dout)],
        out_specs=pl.BlockSpec((BR, dout), lambda i: (i, 0)),
        out_shape=jax.ShapeDtypeStruct((nrows, dout), jnp.float32),
    )(x, wroot, S, cnt, w1, S, cnt, w2,
      b.reshape(1, dout), g.reshape(1, dout), be.reshape(1, dout))


def _head2_body(x_ref, wroot_ref, s1_ref, c1_ref, w1_ref, s2_ref, c2_ref, w2_ref,
                b_ref, g_ref, be_ref, wc1_ref, bc1_ref, wc2_ref, bc2_ref, o_ref,
                *, two_rel):
    acc = jnp.dot(x_ref[...], wroot_ref[...], preferred_element_type=jnp.float32)
    acc += jnp.dot(_mean(s1_ref, c1_ref), w1_ref[...], preferred_element_type=jnp.float32)
    if two_rel:
        acc += jnp.dot(_mean(s2_ref, c2_ref), w2_ref[...], preferred_element_type=jnp.float32)
    acc += b_ref[...]
    acc = g_ref[...] * acc * _BN_SCALE + be_ref[...]
    h = jnp.maximum(jnp.dot(acc, wc1_ref[...], preferred_element_type=jnp.float32)
                    + bc1_ref[...], 0.0)
    o_ref[...] = jnp.dot(h, wc2_ref[...], preferred_element_type=jnp.float32) + bc2_ref[...]


def _head(nrows, x, xoff, S, s1off, s2off, cnt, wroot, w1, w2, b, g, be,
          wc1, bc1, wc2p, bc2p, two_rel):
    body = functools.partial(_head2_body, two_rel=two_rel)
    return pl.pallas_call(
        body,
        grid=(nrows // BR,),
        in_specs=[_row_spec(xoff, 128), _full_spec(128, 64),
                  _row_spec(s1off, 128), _row_spec(s1off, 128), _full_spec(128, 64),
                  _row_spec(s2off, 128), _row_spec(s2off, 128), _full_spec(128, 64),
                  _full_spec(1, 64), _full_spec(1, 64), _full_spec(1, 64),
                  _full_spec(64, 64), _full_spec(1, 64),
                  _full_spec(64, 128), _full_spec(1, 128)],
        out_specs=pl.BlockSpec((BR, 128), lambda i: (i, 0)),
        out_shape=jax.ShapeDtypeStruct((nrows, 128), jnp.float32),
    )(x, wroot, S, cnt, w1, S, cnt, w2,
      b.reshape(1, 64), g.reshape(1, 64), be.reshape(1, 64),
      wc1, bc1.reshape(1, 64), wc2p, bc2p.reshape(1, 128))


def _pad1(a, n, val):
    return jnp.concatenate([a.astype(jnp.int32),
                            jnp.full((n - a.shape[0],), val, jnp.int32)])


def _slot_arrays(eip, eir, eit):
    s_p = _pad1(eip[0], E0P, 0)
    d_p = _pad1(eip[1], E0P, GR)
    s_r = _pad1(eir[0] + N_ACC, E0P, 0)
    d_r = _pad1(eir[1], E0P, GR)
    s_t = _pad1(eit[0], E2P, 0)
    dt = _pad1(eit[1], E2P, -1)

    def chunk_dst(lo):
        inr = (dt >= lo) & (dt < lo + 10000)
        return jnp.where(inr, dt - lo, GR)

    chunks = tuple(chunk_dst(k * 10000) for k in range(5))
    return s_p, d_p, s_r, d_r, s_t, chunks


def kernel(x_account, x_merchant, edge_index_pays, edge_index_receives,
           edge_index_transfers, Wp_account, bp_account, Wp_merchant, bp_merchant,
           Wrel0, Wroot0, b0, gamma0, beta0,
           Wrel1, Wroot1, b1, gamma1, beta1,
           Wrel2, Wroot2, b2, gamma2, beta2,
           Wc1, bc1, Wc2, bc2):
    s_p, d_p, s_r, d_r, s_t, dchunks = _slot_arrays(
        edge_index_pays, edge_index_receives, edge_index_transfers)
    zeros_hbm = jnp.zeros((ZROWS, 128), jnp.float32)
    ones_hbm = jnp.ones((CHUNK, 128), jnp.float32)

    cnt = _sc_counts_pass_k()(d_p, d_r, *dchunks, zeros_hbm, ones_hbm)
    feature_pass = _sc_feature_pass_k()

    ha = _tc_proj(x_account, Wp_account, bp_account)
    hm = _tc_proj(x_merchant, Wp_merchant, bp_merchant)
    x = jnp.concatenate([ha, hm], axis=0)

    layers = ((Wrel0, Wroot0, b0, gamma0, beta0),
              (Wrel1, Wroot1, b1, gamma1, beta1))
    for (Wrel, Wroot, b, g, be) in layers:
        S = feature_pass(x, s_p, d_p, s_r, d_r, s_t, *dchunks, zeros_hbm)
        pA = _combine2(10000, x, 0, S, 10000, 20000, cnt, Wroot,
                       Wrel[1], Wrel[2], b, g, be, True)
        pB = _combine1(40000, x, 10000, S, 30000, cnt, Wroot,
                       Wrel[2], b, g, be, True)
        pC = _combine1(10000, x, 50000, S, 0, cnt, Wroot,
                       Wrel[0], b, g, be, True)
        x = jnp.concatenate([pA, pB, pC], axis=0)

    S = feature_pass(x, s_p, d_p, s_r, d_r, s_t, *dchunks, zeros_hbm)
    wc2p = jnp.zeros((64, 128), jnp.float32).at[:, :2].set(Wc2)
    bc2p = jnp.zeros((128,), jnp.float32).at[:2].set(bc2)
    lA = _head(10000, x, 0, S, 10000, 20000, cnt, Wroot2,
               Wrel2[1], Wrel2[2], b2, gamma2, beta2, Wc1, bc1, wc2p, bc2p, True)
    lB = _head(40000, x, 10000, S, 30000, 30000, cnt, Wroot2,
               Wrel2[2], Wrel2[2], b2, gamma2, beta2, Wc1, bc1, wc2p, bc2p, False)
    logits = jnp.concatenate([lA, lB], axis=0)[:, :2]
    return logits


# speedup vs baseline: 7.2104x; 1.3588x over previous
import functools
import jax
import jax.numpy as jnp
from jax import lax
from jax.experimental import pallas as pl
from jax.experimental.pallas import tpu as pltpu
from jax.experimental.pallas import tpu_sc as plsc

N_ACC = 50000
N_MER = 10000
N = N_ACC + N_MER

NTILES = 16
CHUNK = 112
ACC_ROWS = 12544
GR = 12520
ZROWS = ACC_ROWS // NTILES

E0P = 250880
E2P = 100352
NC0 = E0P // NTILES // CHUNK
NC2 = E2P // NTILES // CHUNK

CB = (0, 12504, 25008, 37512)
CW = (12504, 12504, 12504, 12488)

def _sc_writeout(acc, out_hbm, s, ob, valid):
    per = (valid // NTILES) & ~7
    rem = valid - per * NTILES
    pltpu.sync_copy(acc.at[pl.ds(s * per, per)], out_hbm.at[pl.ds(ob + s * per, per)])
    @pl.when(s == 0)
    def _():
        pltpu.sync_copy(acc.at[pl.ds(per * NTILES, rem)],
                        out_hbm.at[pl.ds(ob + per * NTILES, rem)])


def _run_slot(srcR, dstR, nchunk, ob, valid, x_hbm, out_hbm, zeros_hbm,
              idx2, dst2, rows2, acc, sem, s):
    pltpu.sync_copy(zeros_hbm, acc.at[pl.ds(s * ZROWS, ZROWS)])
    base_e = s * nchunk * CHUNK
    pltpu.sync_copy(srcR.at[pl.ds(base_e, CHUNK)], idx2.at[0])
    pltpu.sync_copy(dstR.at[pl.ds(base_e, CHUNK)], dst2.at[0])
    plsc.subcore_barrier()
    pltpu.async_copy(x_hbm.at[idx2.at[0]], rows2.at[0], sem)

    def body(g, carry):
        p = lax.rem(g, 2)
        q = 1 - p
        pltpu.make_async_copy(x_hbm.at[idx2.at[p]], rows2.at[p], sem).wait()

        @pl.when(g < nchunk - 1)
        def _():
            off = base_e + (g + 1) * CHUNK
            pltpu.sync_copy(srcR.at[pl.ds(off, CHUNK)], idx2.at[q])
            pltpu.sync_copy(dstR.at[pl.ds(off, CHUNK)], dst2.at[q])
            pltpu.async_copy(x_hbm.at[idx2.at[q]], rows2.at[q], sem)

        pltpu.sync_copy(rows2.at[p], acc.at[dst2.at[p]], add=True)
        return carry

    lax.fori_loop(0, nchunk, body, 0)
    plsc.subcore_barrier()
    _sc_writeout(acc, out_hbm, s, ob, valid)
    plsc.subcore_barrier()


@functools.lru_cache(maxsize=None)
def _sc_feature_pass_k():
    mesh = plsc.VectorSubcoreMesh(core_axis_name="c", subcore_axis_name="s")

    @functools.partial(
        pl.kernel, mesh=mesh,
        out_type=jax.ShapeDtypeStruct((70000, 128), jnp.float32),
        scratch_types=[
            pltpu.VMEM((2, CHUNK), jnp.int32),
            pltpu.VMEM((2, CHUNK), jnp.int32),
            pltpu.VMEM((2, CHUNK, 128), jnp.float32),
            pltpu.VMEM_SHARED((ACC_ROWS, 128), jnp.float32),
            pltpu.SemaphoreType.DMA,
        ])
    def _sc_feature_pass(x_hbm, src_p, dst_p, src_r, dst_r, src_t,
                         dst_c0, dst_c1, dst_c2, dst_c3, zeros_hbm,
                         out_hbm, idx2, dst2, rows2, acc, sem):
        c = lax.axis_index("c")
        s = lax.axis_index("s")
        core_slots = (
            ((src_p, dst_p, NC0, 0, 10000),
             (src_t, dst_c0, NC2, 20000, CW[0]),
             (src_t, dst_c1, NC2, 20000 + CB[1], CW[1])),
            ((src_r, dst_r, NC0, 10000, 10000),
             (src_t, dst_c2, NC2, 20000 + CB[2], CW[2]),
             (src_t, dst_c3, NC2, 20000 + CB[3], CW[3])),
        )
        for core_id in (0, 1):
            @pl.when(c == core_id)
            def _():
                for (srcR, dstR, nchunk, ob, valid) in core_slots[core_id]:
                    _run_slot(srcR, dstR, nchunk, ob, valid, x_hbm, out_hbm,
                              zeros_hbm, idx2, dst2, rows2, acc, sem, s)

    return _sc_feature_pass


def _run_cslot(dstR, nchunk, ob, valid, out_hbm, zerosc_hbm,
               dst_v, ones_v, acc, sem, s):
    pltpu.sync_copy(zerosc_hbm, acc.at[pl.ds(s * ZROWS, ZROWS)])
    plsc.subcore_barrier()
    base_e = s * nchunk * CHUNK

    def body(g, carry):
        pltpu.sync_copy(dstR.at[pl.ds(base_e + g * CHUNK, CHUNK)], dst_v)
        pltpu.sync_copy(ones_v, acc.at[dst_v], add=True)
        return carry

    lax.fori_loop(0, nchunk, body, 0)
    plsc.subcore_barrier()
    _sc_writeout(acc, out_hbm, s, ob, valid)
    plsc.subcore_barrier()


@functools.lru_cache(maxsize=None)
def _sc_counts_pass_k():
    mesh = plsc.VectorSubcoreMesh(core_axis_name="c", subcore_axis_name="s")

    @functools.partial(
        pl.kernel, mesh=mesh,
        out_type=jax.ShapeDtypeStruct((70000, 128), jnp.float32),
        scratch_types=[
            pltpu.VMEM((CHUNK,), jnp.int32),
            pltpu.VMEM((CHUNK, 128), jnp.float32),
            pltpu.VMEM_SHARED((ACC_ROWS, 128), jnp.float32),
            pltpu.SemaphoreType.DMA,
        ])
    def _sc_counts_pass(dst_p, dst_r, dst_c0, dst_c1, dst_c2, dst_c3,
                        zerosc_hbm, ones_hbm,
                        out_hbm, dst_v, ones_v, acc, sem):
        c = lax.axis_index("c")
        s = lax.axis_index("s")
        pltpu.sync_copy(ones_hbm, ones_v)
        core_slots = (
            ((dst_p, NC0, 0, 10000),
             (dst_c0, NC2, 20000, CW[0]),
             (dst_c1, NC2, 20000 + CB[1], CW[1])),
            ((dst_r, NC0, 10000, 10000),
             (dst_c2, NC2, 20000 + CB[2], CW[2]),
             (dst_c3, NC2, 20000 + CB[3], CW[3])),
        )
        for core_id in (0, 1):
            @pl.when(c == core_id)
            def _():
                for (dstR, nchunk, ob, valid) in core_slots[core_id]:
                    _run_cslot(dstR, nchunk, ob, valid, out_hbm, zerosc_hbm,
                               dst_v, ones_v, acc, sem, s)

    return _sc_counts_pass


BR = 2000


def _proj_body(x_ref, w_ref, b_ref, o_ref):
    o_ref[...] = jnp.dot(x_ref[...], w_ref[...],
                         preferred_element_type=jnp.float32) + b_ref[...]


def _tc_proj(x, w, b):
    n = x.shape[0]
    return pl.pallas_call(
        _proj_body,
        grid=(n // BR,),
        in_specs=[pl.BlockSpec((BR, 128), lambda i: (i, 0)),
                  pl.BlockSpec((128, 128), lambda i: (0, 0)),
                  pl.BlockSpec((1, 128), lambda i: (0, 0))],
        out_specs=pl.BlockSpec((BR, 128), lambda i: (i, 0)),
        out_shape=jax.ShapeDtypeStruct((n, 128), jnp.float32),
    )(x, w, b.reshape(1, 128))


_BN_SCALE = 1.0 / (1.0 + 1e-5) ** 0.5


def _mean(s_ref, c_ref):
    inv = 1.0 / jnp.maximum(c_ref[...][:, 0:1], 1.0)
    return s_ref[...] * inv


def _combine1_body(x_ref, wroot_ref, s_ref, c_ref, w_ref, b_ref, g_ref, be_ref,
                   o_ref, *, relu):
    acc = jnp.dot(x_ref[...], wroot_ref[...], preferred_element_type=jnp.float32)
    acc += jnp.dot(_mean(s_ref, c_ref), w_ref[...], preferred_element_type=jnp.float32)
    acc += b_ref[...]
    acc = g_ref[...] * acc * _BN_SCALE + be_ref[...]
    o_ref[...] = jnp.maximum(acc, 0.0) if relu else acc


def _combine2_body(x_ref, wroot_ref, s1_ref, c1_ref, w1_ref, s2_ref, c2_ref, w2_ref,
                   b_ref, g_ref, be_ref, o_ref, *, relu):
    acc = jnp.dot(x_ref[...], wroot_ref[...], preferred_element_type=jnp.float32)
    acc += jnp.dot(_mean(s1_ref, c1_ref), w1_ref[...], preferred_element_type=jnp.float32)
    acc += jnp.dot(_mean(s2_ref, c2_ref), w2_ref[...], preferred_element_type=jnp.float32)
    acc += b_ref[...]
    acc = g_ref[...] * acc * _BN_SCALE + be_ref[...]
    o_ref[...] = jnp.maximum(acc, 0.0) if relu else acc


def _row_spec(off, w):
    o = off // BR
    return pl.BlockSpec((BR, w), lambda i, o=o: (i + o, 0))


def _full_spec(r, cdim):
    return pl.BlockSpec((r, cdim), lambda i: (0, 0))


def _combine1(nrows, x, xoff, S, soff, cnt, wroot, w, b, g, be, relu):
    dout = w.shape[1]
    body = functools.partial(_combine1_body, relu=relu)
    return pl.pallas_call(
        body,
        grid=(nrows // BR,),
        in_specs=[_row_spec(xoff, 128), _full_spec(128, dout),
                  _row_spec(soff, 128), _row_spec(soff, 128), _full_spec(128, dout),
                  _full_spec(1, dout), _full_spec(1, dout), _full_spec(1, dout)],
        out_specs=pl.BlockSpec((BR, dout), lambda i: (i, 0)),
        out_shape=jax.ShapeDtypeStruct((nrows, dout), jnp.float32),
    )(x, wroot, S, cnt, w, b.reshape(1, dout), g.reshape(1, dout), be.reshape(1, dout))


def _combine2(nrows, x, xoff, S, s1off, s2off, cnt, wroot, w1, w2, b, g, be, relu):
    dout = w1.shape[1]
    body = functools.partial(_combine2_body, relu=relu)
    return pl.pallas_call(
        body,
        grid=(nrows // BR,),
        in_specs=[_row_spec(xoff, 128), _full_spec(128, dout),
                  _row_spec(s1off, 128), _row_spec(s1off, 128), _full_spec(128, dout),
                  _row_spec(s2off, 128), _row_spec(s2off, 128), _full_spec(128, dout),
                  _full_spec(1, dout), _full_spec(1, dout), _full_spec(1, dout)],
        out_specs=pl.BlockSpec((BR, dout), lambda i: (i, 0)),
        out_shape=jax.ShapeDtypeStruct((nrows, dout), jnp.float32),
    )(x, wroot, S, cnt, w1, S, cnt, w2,
      b.reshape(1, dout), g.reshape(1, dout), be.reshape(1, dout))


def _head2_body(x_ref, wroot_ref, s1_ref, c1_ref, w1_ref, s2_ref, c2_ref, w2_ref,
                b_ref, g_ref, be_ref, wc1_ref, bc1_ref, wc2_ref, bc2_ref, o_ref,
                *, two_rel):
    acc = jnp.dot(x_ref[...], wroot_ref[...], preferred_element_type=jnp.float32)
    acc += jnp.dot(_mean(s1_ref, c1_ref), w1_ref[...], preferred_element_type=jnp.float32)
    if two_rel:
        acc += jnp.dot(_mean(s2_ref, c2_ref), w2_ref[...], preferred_element_type=jnp.float32)
    acc += b_ref[...]
    acc = g_ref[...] * acc * _BN_SCALE + be_ref[...]
    h = jnp.maximum(jnp.dot(acc, wc1_ref[...], preferred_element_type=jnp.float32)
                    + bc1_ref[...], 0.0)
    o_ref[...] = jnp.dot(h, wc2_ref[...], preferred_element_type=jnp.float32) + bc2_ref[...]


def _head(nrows, x, xoff, S, s1off, s2off, cnt, wroot, w1, w2, b, g, be,
          wc1, bc1, wc2p, bc2p, two_rel):
    body = functools.partial(_head2_body, two_rel=two_rel)
    return pl.pallas_call(
        body,
        grid=(nrows // BR,),
        in_specs=[_row_spec(xoff, 128), _full_spec(128, 64),
                  _row_spec(s1off, 128), _row_spec(s1off, 128), _full_spec(128, 64),
                  _row_spec(s2off, 128), _row_spec(s2off, 128), _full_spec(128, 64),
                  _full_spec(1, 64), _full_spec(1, 64), _full_spec(1, 64),
                  _full_spec(64, 64), _full_spec(1, 64),
                  _full_spec(64, 128), _full_spec(1, 128)],
        out_specs=pl.BlockSpec((BR, 128), lambda i: (i, 0)),
        out_shape=jax.ShapeDtypeStruct((nrows, 128), jnp.float32),
    )(x, wroot, S, cnt, w1, S, cnt, w2,
      b.reshape(1, 64), g.reshape(1, 64), be.reshape(1, 64),
      wc1, bc1.reshape(1, 64), wc2p, bc2p.reshape(1, 128))


def _pad1(a, n, val):
    return jnp.concatenate([a.astype(jnp.int32),
                            jnp.full((n - a.shape[0],), val, jnp.int32)])


def _slot_arrays(eip, eir, eit):
    s_p = _pad1(eip[0], E0P, 0)
    d_p = _pad1(eip[1], E0P, GR)
    s_r = _pad1(eir[0] + N_ACC, E0P, 0)
    d_r = _pad1(eir[1], E0P, GR)
    s_t = _pad1(eit[0], E2P, 0)
    dt = _pad1(eit[1], E2P, -1)

    def chunk_dst(k):
        inr = (dt >= CB[k]) & (dt < CB[k] + CW[k])
        return jnp.where(inr, dt - CB[k], GR)

    chunks = tuple(chunk_dst(k) for k in range(4))
    return s_p, d_p, s_r, d_r, s_t, chunks


def kernel(x_account, x_merchant, edge_index_pays, edge_index_receives,
           edge_index_transfers, Wp_account, bp_account, Wp_merchant, bp_merchant,
           Wrel0, Wroot0, b0, gamma0, beta0,
           Wrel1, Wroot1, b1, gamma1, beta1,
           Wrel2, Wroot2, b2, gamma2, beta2,
           Wc1, bc1, Wc2, bc2):
    s_p, d_p, s_r, d_r, s_t, dchunks = _slot_arrays(
        edge_index_pays, edge_index_receives, edge_index_transfers)
    zeros_hbm = jnp.zeros((ZROWS, 128), jnp.float32)
    ones_hbm = jnp.ones((CHUNK, 128), jnp.float32)

    cnt = _sc_counts_pass_k()(d_p, d_r, *dchunks, zeros_hbm, ones_hbm)
    feature_pass = _sc_feature_pass_k()

    ha = _tc_proj(x_account, Wp_account, bp_account)
    hm = _tc_proj(x_merchant, Wp_merchant, bp_merchant)
    x = jnp.concatenate([ha, hm], axis=0)

    layers = ((Wrel0, Wroot0, b0, gamma0, beta0),
              (Wrel1, Wroot1, b1, gamma1, beta1))
    for (Wrel, Wroot, b, g, be) in layers:
        S = feature_pass(x, s_p, d_p, s_r, d_r, s_t, *dchunks, zeros_hbm)
        pA = _combine2(10000, x, 0, S, 10000, 20000, cnt, Wroot,
                       Wrel[1], Wrel[2], b, g, be, True)
        pB = _combine1(40000, x, 10000, S, 30000, cnt, Wroot,
                       Wrel[2], b, g, be, True)
        pC = _combine1(10000, x, 50000, S, 0, cnt, Wroot,
                       Wrel[0], b, g, be, True)
        x = jnp.concatenate([pA, pB, pC], axis=0)

    S = feature_pass(x, s_p, d_p, s_r, d_r, s_t, *dchunks, zeros_hbm)
    wc2p = jnp.zeros((64, 128), jnp.float32).at[:, :2].set(Wc2)
    bc2p = jnp.zeros((128,), jnp.float32).at[:2].set(bc2)
    lA = _head(10000, x, 0, S, 10000, 20000, cnt, Wroot2,
               Wrel2[1], Wrel2[2], b2, gamma2, beta2, Wc1, bc1, wc2p, bc2p, True)
    lB = _head(40000, x, 10000, S, 30000, 30000, cnt, Wroot2,
               Wrel2[2], Wrel2[2], b2, gamma2, beta2, Wc1, bc1, wc2p, bc2p, False)
    logits = jnp.concatenate([lA, lB], axis=0)[:, :2]
    return logits


# speedup vs baseline: 7.3671x; 1.0217x over previous
import functools
import jax
import jax.numpy as jnp
from jax import lax
from jax.experimental import pallas as pl
from jax.experimental.pallas import tpu as pltpu
from jax.experimental.pallas import tpu_sc as plsc

N_ACC = 50000
N_MER = 10000
N = N_ACC + N_MER

NTILES = 16
CHUNK = 112
ACC_ROWS = 12544
GR = 12520
ZROWS = ACC_ROWS // NTILES

E0P = 250880
E2P = 100352
NC0 = E0P // NTILES // CHUNK
NC2 = E2P // NTILES // CHUNK

CB = (0, 12504, 25008, 37512)
CW = (12504, 12504, 12504, 12488)

def _sc_writeout(acc, out_hbm, s, ob, valid):
    per = (valid // NTILES) & ~7
    rem = valid - per * NTILES
    pltpu.sync_copy(acc.at[pl.ds(s * per, per)], out_hbm.at[pl.ds(ob + s * per, per)])
    @pl.when(s == 0)
    def _():
        pltpu.sync_copy(acc.at[pl.ds(per * NTILES, rem)],
                        out_hbm.at[pl.ds(ob + per * NTILES, rem)])


def _run_slot(srcR, dstR, nchunk, ob, valid, x_hbm, out_hbm, zeros_hbm,
              idx2, dst2, rows2, acc, sem, sem2, s):
    pltpu.sync_copy(zeros_hbm, acc.at[pl.ds(s * ZROWS, ZROWS)])
    base_e = s * nchunk * CHUNK
    pltpu.sync_copy(srcR.at[pl.ds(base_e, CHUNK)], idx2.at[0])
    pltpu.sync_copy(dstR.at[pl.ds(base_e, CHUNK)], dst2.at[0])
    plsc.subcore_barrier()
    pltpu.async_copy(x_hbm.at[idx2.at[0]], rows2.at[0], sem)

    def body(g, carry):
        p = lax.rem(g, 2)
        q = 1 - p
        pltpu.make_async_copy(x_hbm.at[idx2.at[p]], rows2.at[p], sem).wait()

        @pl.when(g >= 1)
        def _():
            pltpu.make_async_copy(rows2.at[q], acc.at[dst2.at[q]], sem2).wait()

        @pl.when(g < nchunk - 1)
        def _():
            off = base_e + (g + 1) * CHUNK
            pltpu.sync_copy(srcR.at[pl.ds(off, CHUNK)], idx2.at[q])
            pltpu.sync_copy(dstR.at[pl.ds(off, CHUNK)], dst2.at[q])
            pltpu.async_copy(x_hbm.at[idx2.at[q]], rows2.at[q], sem)

        pltpu.async_copy(rows2.at[p], acc.at[dst2.at[p]], sem2, add=True)
        return carry

    lax.fori_loop(0, nchunk, body, 0)
    pltpu.make_async_copy(rows2.at[1], acc.at[dst2.at[1]], sem2).wait()
    plsc.subcore_barrier()
    _sc_writeout(acc, out_hbm, s, ob, valid)
    plsc.subcore_barrier()


@functools.lru_cache(maxsize=None)
def _sc_feature_pass_k():
    mesh = plsc.VectorSubcoreMesh(core_axis_name="c", subcore_axis_name="s")

    @functools.partial(
        pl.kernel, mesh=mesh,
        out_type=jax.ShapeDtypeStruct((70000, 128), jnp.float32),
        scratch_types=[
            pltpu.VMEM((2, CHUNK), jnp.int32),
            pltpu.VMEM((2, CHUNK), jnp.int32),
            pltpu.VMEM((2, CHUNK, 128), jnp.float32),
            pltpu.VMEM_SHARED((ACC_ROWS, 128), jnp.float32),
            pltpu.SemaphoreType.DMA,
            pltpu.SemaphoreType.DMA,
        ])
    def _sc_feature_pass(x_hbm, src_p, dst_p, src_r, dst_r, src_t,
                         dst_c0, dst_c1, dst_c2, dst_c3, zeros_hbm,
                         out_hbm, idx2, dst2, rows2, acc, sem, sem2):
        c = lax.axis_index("c")
        s = lax.axis_index("s")
        core_slots = (
            ((src_p, dst_p, NC0, 0, 10000),
             (src_t, dst_c0, NC2, 20000, CW[0]),
             (src_t, dst_c1, NC2, 20000 + CB[1], CW[1])),
            ((src_r, dst_r, NC0, 10000, 10000),
             (src_t, dst_c2, NC2, 20000 + CB[2], CW[2]),
             (src_t, dst_c3, NC2, 20000 + CB[3], CW[3])),
        )
        for core_id in (0, 1):
            @pl.when(c == core_id)
            def _():
                for (srcR, dstR, nchunk, ob, valid) in core_slots[core_id]:
                    _run_slot(srcR, dstR, nchunk, ob, valid, x_hbm, out_hbm,
                              zeros_hbm, idx2, dst2, rows2, acc, sem, sem2, s)

    return _sc_feature_pass


def _run_cslot(dstR, nchunk, ob, valid, out_hbm, zerosc_hbm,
               dst2, ones_v, acc, sem, s):
    pltpu.sync_copy(zerosc_hbm, acc.at[pl.ds(s * ZROWS, ZROWS)])
    plsc.subcore_barrier()
    base_e = s * nchunk * CHUNK

    def body(g, carry):
        p = lax.rem(g, 2)

        @pl.when(g >= 2)
        def _():
            pltpu.make_async_copy(ones_v, acc.at[dst2.at[p]], sem).wait()

        pltpu.sync_copy(dstR.at[pl.ds(base_e + g * CHUNK, CHUNK)], dst2.at[p])
        pltpu.async_copy(ones_v, acc.at[dst2.at[p]], sem, add=True)
        return carry

    lax.fori_loop(0, nchunk, body, 0)
    pltpu.make_async_copy(ones_v, acc.at[dst2.at[0]], sem).wait()
    pltpu.make_async_copy(ones_v, acc.at[dst2.at[1]], sem).wait()
    plsc.subcore_barrier()
    _sc_writeout(acc, out_hbm, s, ob, valid)
    plsc.subcore_barrier()


@functools.lru_cache(maxsize=None)
def _sc_counts_pass_k():
    mesh = plsc.VectorSubcoreMesh(core_axis_name="c", subcore_axis_name="s")

    @functools.partial(
        pl.kernel, mesh=mesh,
        out_type=jax.ShapeDtypeStruct((70000, 128), jnp.float32),
        scratch_types=[
            pltpu.VMEM((2, CHUNK), jnp.int32),
            pltpu.VMEM((CHUNK, 128), jnp.float32),
            pltpu.VMEM_SHARED((ACC_ROWS, 128), jnp.float32),
            pltpu.SemaphoreType.DMA,
        ])
    def _sc_counts_pass(dst_p, dst_r, dst_c0, dst_c1, dst_c2, dst_c3,
                        zerosc_hbm, ones_hbm,
                        out_hbm, dst2, ones_v, acc, sem):
        c = lax.axis_index("c")
        s = lax.axis_index("s")
        pltpu.sync_copy(ones_hbm, ones_v)
        core_slots = (
            ((dst_p, NC0, 0, 10000),
             (dst_c0, NC2, 20000, CW[0]),
             (dst_c1, NC2, 20000 + CB[1], CW[1])),
            ((dst_r, NC0, 10000, 10000),
             (dst_c2, NC2, 20000 + CB[2], CW[2]),
             (dst_c3, NC2, 20000 + CB[3], CW[3])),
        )
        for core_id in (0, 1):
            @pl.when(c == core_id)
            def _():
                for (dstR, nchunk, ob, valid) in core_slots[core_id]:
                    _run_cslot(dstR, nchunk, ob, valid, out_hbm, zerosc_hbm,
                               dst2, ones_v, acc, sem, s)

    return _sc_counts_pass


BR = 2000


def _proj_body(x_ref, w_ref, b_ref, o_ref):
    o_ref[...] = jnp.dot(x_ref[...], w_ref[...],
                         preferred_element_type=jnp.float32) + b_ref[...]


def _tc_proj(x, w, b):
    n = x.shape[0]
    return pl.pallas_call(
        _proj_body,
        grid=(n // BR,),
        in_specs=[pl.BlockSpec((BR, 128), lambda i: (i, 0)),
                  pl.BlockSpec((128, 128), lambda i: (0, 0)),
                  pl.BlockSpec((1, 128), lambda i: (0, 0))],
        out_specs=pl.BlockSpec((BR, 128), lambda i: (i, 0)),
        out_shape=jax.ShapeDtypeStruct((n, 128), jnp.float32),
    )(x, w, b.reshape(1, 128))


_BN_SCALE = 1.0 / (1.0 + 1e-5) ** 0.5


def _mean(s_ref, c_ref):
    inv = 1.0 / jnp.maximum(c_ref[...][:, 0:1], 1.0)
    return s_ref[...] * inv


def _combine1_body(x_ref, wroot_ref, s_ref, c_ref, w_ref, b_ref, g_ref, be_ref,
                   o_ref, *, relu):
    acc = jnp.dot(x_ref[...], wroot_ref[...], preferred_element_type=jnp.float32)
    acc += jnp.dot(_mean(s_ref, c_ref), w_ref[...], preferred_element_type=jnp.float32)
    acc += b_ref[...]
    acc = g_ref[...] * acc * _BN_SCALE + be_ref[...]
    o_ref[...] = jnp.maximum(acc, 0.0) if relu else acc


def _combine2_body(x_ref, wroot_ref, s1_ref, c1_ref, w1_ref, s2_ref, c2_ref, w2_ref,
                   b_ref, g_ref, be_ref, o_ref, *, relu):
    acc = jnp.dot(x_ref[...], wroot_ref[...], preferred_element_type=jnp.float32)
    acc += jnp.dot(_mean(s1_ref, c1_ref), w1_ref[...], preferred_element_type=jnp.float32)
    acc += jnp.dot(_mean(s2_ref, c2_ref), w2_ref[...], preferred_element_type=jnp.float32)
    acc += b_ref[...]
    acc = g_ref[...] * acc * _BN_SCALE + be_ref[...]
    o_ref[...] = jnp.maximum(acc, 0.0) if relu else acc


def _row_spec(off, w):
    o = off // BR
    return pl.BlockSpec((BR, w), lambda i, o=o: (i + o, 0))


def _full_spec(r, cdim):
    return pl.BlockSpec((r, cdim), lambda i: (0, 0))


def _combine1(nrows, x, xoff, S, soff, cnt, wroot, w, b, g, be, relu):
    dout = w.shape[1]
    body = functools.partial(_combine1_body, relu=relu)
    return pl.pallas_call(
        body,
        grid=(nrows // BR,),
        in_specs=[_row_spec(xoff, 128), _full_spec(128, dout),
                  _row_spec(soff, 128), _row_spec(soff, 128), _full_spec(128, dout),
                  _full_spec(1, dout), _full_spec(1, dout), _full_spec(1, dout)],
        out_specs=pl.BlockSpec((BR, dout), lambda i: (i, 0)),
        out_shape=jax.ShapeDtypeStruct((nrows, dout), jnp.float32),
    )(x, wroot, S, cnt, w, b.reshape(1, dout), g.reshape(1, dout), be.reshape(1, dout))


def _combine2(nrows, x, xoff, S, s1off, s2off, cnt, wroot, w1, w2, b, g, be, relu):
    dout = w1.shape[1]
    body = functools.partial(_combine2_body, relu=relu)
    return pl.pallas_call(
        body,
        grid=(nrows // BR,),
        in_specs=[_row_spec(xoff, 128), _full_spec(128, dout),
                  _row_spec(s1off, 128), _row_spec(s1off, 128), _full_spec(128, dout),
                  _row_spec(s2off, 128), _row_spec(s2off, 128), _full_spec(128, dout),
                  _full_spec(1, dout), _full_spec(1, dout), _full_spec(1, dout)],
        out_specs=pl.BlockSpec((BR, dout), lambda i: (i, 0)),
        out_shape=jax.ShapeDtypeStruct((nrows, dout), jnp.float32),
    )(x, wroot, S, cnt, w1, S, cnt, w2,
      b.reshape(1, dout), g.reshape(1, dout), be.reshape(1, dout))


def _head2_body(x_ref, wroot_ref, s1_ref, c1_ref, w1_ref, s2_ref, c2_ref, w2_ref,
                b_ref, g_ref, be_ref, wc1_ref, bc1_ref, wc2_ref, bc2_ref, o_ref,
                *, two_rel):
    acc = jnp.dot(x_ref[...], wroot_ref[...], preferred_element_type=jnp.float32)
    acc += jnp.dot(_mean(s1_ref, c1_ref), w1_ref[...], preferred_element_type=jnp.float32)
    if two_rel:
        acc += jnp.dot(_mean(s2_ref, c2_ref), w2_ref[...], preferred_element_type=jnp.float32)
    acc += b_ref[...]
    acc = g_ref[...] * acc * _BN_SCALE + be_ref[...]
    h = jnp.maximum(jnp.dot(acc, wc1_ref[...], preferred_element_type=jnp.float32)
                    + bc1_ref[...], 0.0)
    o_ref[...] = jnp.dot(h, wc2_ref[...], preferred_element_type=jnp.float32) + bc2_ref[...]


def _head(nrows, x, xoff, S, s1off, s2off, cnt, wroot, w1, w2, b, g, be,
          wc1, bc1, wc2p, bc2p, two_rel):
    body = functools.partial(_head2_body, two_rel=two_rel)
    return pl.pallas_call(
        body,
        grid=(nrows // BR,),
        in_specs=[_row_spec(xoff, 128), _full_spec(128, 64),
                  _row_spec(s1off, 128), _row_spec(s1off, 128), _full_spec(128, 64),
                  _row_spec(s2off, 128), _row_spec(s2off, 128), _full_spec(128, 64),
                  _full_spec(1, 64), _full_spec(1, 64), _full_spec(1, 64),
                  _full_spec(64, 64), _full_spec(1, 64),
                  _full_spec(64, 128), _full_spec(1, 128)],
        out_specs=pl.BlockSpec((BR, 128), lambda i: (i, 0)),
        out_shape=jax.ShapeDtypeStruct((nrows, 128), jnp.float32),
    )(x, wroot, S, cnt, w1, S, cnt, w2,
      b.reshape(1, 64), g.reshape(1, 64), be.reshape(1, 64),
      wc1, bc1.reshape(1, 64), wc2p, bc2p.reshape(1, 128))


def _pad1(a, n, val):
    return jnp.concatenate([a.astype(jnp.int32),
                            jnp.full((n - a.shape[0],), val, jnp.int32)])


def _slot_arrays(eip, eir, eit):
    s_p = _pad1(eip[0], E0P, 0)
    d_p = _pad1(eip[1], E0P, GR)
    s_r = _pad1(eir[0] + N_ACC, E0P, 0)
    d_r = _pad1(eir[1], E0P, GR)
    s_t = _pad1(eit[0], E2P, 0)
    dt = _pad1(eit[1], E2P, -1)

    def chunk_dst(k):
        inr = (dt >= CB[k]) & (dt < CB[k] + CW[k])
        return jnp.where(inr, dt - CB[k], GR)

    chunks = tuple(chunk_dst(k) for k in range(4))
    return s_p, d_p, s_r, d_r, s_t, chunks


def kernel(x_account, x_merchant, edge_index_pays, edge_index_receives,
           edge_index_transfers, Wp_account, bp_account, Wp_merchant, bp_merchant,
           Wrel0, Wroot0, b0, gamma0, beta0,
           Wrel1, Wroot1, b1, gamma1, beta1,
           Wrel2, Wroot2, b2, gamma2, beta2,
           Wc1, bc1, Wc2, bc2):
    s_p, d_p, s_r, d_r, s_t, dchunks = _slot_arrays(
        edge_index_pays, edge_index_receives, edge_index_transfers)
    zeros_hbm = jnp.zeros((ZROWS, 128), jnp.float32)
    ones_hbm = jnp.ones((CHUNK, 128), jnp.float32)

    cnt = _sc_counts_pass_k()(d_p, d_r, *dchunks, zeros_hbm, ones_hbm)
    feature_pass = _sc_feature_pass_k()

    ha = _tc_proj(x_account, Wp_account, bp_account)
    hm = _tc_proj(x_merchant, Wp_merchant, bp_merchant)
    x = jnp.concatenate([ha, hm], axis=0)

    layers = ((Wrel0, Wroot0, b0, gamma0, beta0),
              (Wrel1, Wroot1, b1, gamma1, beta1))
    for (Wrel, Wroot, b, g, be) in layers:
        S = feature_pass(x, s_p, d_p, s_r, d_r, s_t, *dchunks, zeros_hbm)
        pA = _combine2(10000, x, 0, S, 10000, 20000, cnt, Wroot,
                       Wrel[1], Wrel[2], b, g, be, True)
        pB = _combine1(40000, x, 10000, S, 30000, cnt, Wroot,
                       Wrel[2], b, g, be, True)
        pC = _combine1(10000, x, 50000, S, 0, cnt, Wroot,
                       Wrel[0], b, g, be, True)
        x = jnp.concatenate([pA, pB, pC], axis=0)

    S = feature_pass(x, s_p, d_p, s_r, d_r, s_t, *dchunks, zeros_hbm)
    wc2p = jnp.zeros((64, 128), jnp.float32).at[:, :2].set(Wc2)
    bc2p = jnp.zeros((128,), jnp.float32).at[:2].set(bc2)
    lA = _head(10000, x, 0, S, 10000, 20000, cnt, Wroot2,
               Wrel2[1], Wrel2[2], b2, gamma2, beta2, Wc1, bc1, wc2p, bc2p, True)
    lB = _head(40000, x, 10000, S, 30000, 30000, cnt, Wroot2,
               Wrel2[2], Wrel2[2], b2, gamma2, beta2, Wc1, bc1, wc2p, bc2p, False)
    logits = jnp.concatenate([lA, lB], axis=0)[:, :2]
    return logits


# speedup vs baseline: 8.5464x; 1.1601x over previous
import functools
import jax
import jax.numpy as jnp
from jax import lax
from jax.experimental import pallas as pl
from jax.experimental.pallas import tpu as pltpu
from jax.experimental.pallas import tpu_sc as plsc

N_ACC = 50000
N_MER = 10000
N = N_ACC + N_MER

NTILES = 16
CHUNK = 112
ACC_ROWS = 12544
GR = 12520
ZROWS = ACC_ROWS // NTILES

E0P = 250880
E2P = 100352
NC0 = E0P // NTILES // CHUNK
NC2 = E2P // NTILES // CHUNK

CB = (0, 12504, 25008, 37512)
CW = (12504, 12504, 12504, 12488)

def _sc_writeout(acc, out_hbm, s, ob, valid):
    per = (valid // NTILES) & ~7
    rem = valid - per * NTILES
    pltpu.sync_copy(acc.at[pl.ds(s * per, per)], out_hbm.at[pl.ds(ob + s * per, per)])
    @pl.when(s == 0)
    def _():
        pltpu.sync_copy(acc.at[pl.ds(per * NTILES, rem)],
                        out_hbm.at[pl.ds(ob + per * NTILES, rem)])


def _run_slot(packR, nchunk, ob, valid, x_hbm, out_hbm, zeros_hbm,
              grp, rows2, acc, sem, sem2, s):
    ngrp = nchunk // 4
    pltpu.sync_copy(zeros_hbm, acc.at[pl.ds(s * ZROWS, ZROWS)])
    base_row = s * nchunk * 2
    pltpu.sync_copy(packR.at[pl.ds(base_row, 8)], grp.at[0])
    plsc.subcore_barrier()
    pltpu.async_copy(x_hbm.at[grp.at[0, 0]], rows2.at[0], sem)

    def body(G, carry):
        pG = lax.rem(G, 2)
        qG = 1 - pG
        for k in range(4):
            k1 = k & 1
            pltpu.make_async_copy(x_hbm.at[grp.at[pG, 2 * k]],
                                  rows2.at[k1], sem).wait()
            if k == 0:
                @pl.when(G >= 1)
                def _():
                    pltpu.make_async_copy(rows2.at[1 - k1],
                                          acc.at[grp.at[qG, 7]], sem2).wait()

                @pl.when(G < ngrp - 1)
                def _():
                    pltpu.sync_copy(
                        packR.at[pl.ds(base_row + (G + 1) * 8, 8)], grp.at[qG])
            else:
                pltpu.make_async_copy(rows2.at[1 - k1],
                                      acc.at[grp.at[pG, 2 * k - 1]], sem2).wait()
            if k < 3:
                pltpu.async_copy(x_hbm.at[grp.at[pG, 2 * k + 2]],
                                 rows2.at[1 - k1], sem)
            else:
                @pl.when(G < ngrp - 1)
                def _():
                    pltpu.async_copy(x_hbm.at[grp.at[qG, 0]],
                                     rows2.at[1 - k1], sem)
            pltpu.async_copy(rows2.at[k1], acc.at[grp.at[pG, 2 * k + 1]],
                             sem2, add=True)
        return carry

    lax.fori_loop(0, ngrp, body, 0)
    pLast = (ngrp - 1) % 2
    pltpu.make_async_copy(rows2.at[1], acc.at[grp.at[pLast, 7]], sem2).wait()
    plsc.subcore_barrier()
    _sc_writeout(acc, out_hbm, s, ob, valid)
    plsc.subcore_barrier()


@functools.lru_cache(maxsize=None)
def _sc_feature_pass_k():
    mesh = plsc.VectorSubcoreMesh(core_axis_name="c", subcore_axis_name="s")

    @functools.partial(
        pl.kernel, mesh=mesh,
        out_type=jax.ShapeDtypeStruct((70000, 128), jnp.float32),
        scratch_types=[
            pltpu.VMEM((2, 8, CHUNK), jnp.int32),
            pltpu.VMEM((2, CHUNK, 128), jnp.float32),
            pltpu.VMEM_SHARED((ACC_ROWS, 128), jnp.float32),
            pltpu.SemaphoreType.DMA,
            pltpu.SemaphoreType.DMA,
        ])
    def _sc_feature_pass(x_hbm, pk_p, pk_r, pk_c0, pk_c1, pk_c2, pk_c3,
                         zeros_hbm, out_hbm, grp, rows2, acc, sem, sem2):
        c = lax.axis_index("c")
        s = lax.axis_index("s")
        core_slots = (
            ((pk_p, NC0, 0, 10000),
             (pk_c0, NC2, 20000, CW[0]),
             (pk_c1, NC2, 20000 + CB[1], CW[1])),
            ((pk_r, NC0, 10000, 10000),
             (pk_c2, NC2, 20000 + CB[2], CW[2]),
             (pk_c3, NC2, 20000 + CB[3], CW[3])),
        )
        for core_id in (0, 1):
            @pl.when(c == core_id)
            def _():
                for (packR, nchunk, ob, valid) in core_slots[core_id]:
                    _run_slot(packR, nchunk, ob, valid, x_hbm, out_hbm,
                              zeros_hbm, grp, rows2, acc, sem, sem2, s)

    return _sc_feature_pass


def _run_cslot(dstR, nchunk, ob, valid, out_hbm, zerosc_hbm,
               dst2, ones_v, acc, sem, s):
    pltpu.sync_copy(zerosc_hbm, acc.at[pl.ds(s * ZROWS, ZROWS)])
    plsc.subcore_barrier()
    base_e = s * nchunk * CHUNK

    def body(g, carry):
        p = lax.rem(g, 2)

        @pl.when(g >= 2)
        def _():
            pltpu.make_async_copy(ones_v, acc.at[dst2.at[p]], sem).wait()

        pltpu.sync_copy(dstR.at[pl.ds(base_e + g * CHUNK, CHUNK)], dst2.at[p])
        pltpu.async_copy(ones_v, acc.at[dst2.at[p]], sem, add=True)
        return carry

    lax.fori_loop(0, nchunk, body, 0)
    pltpu.make_async_copy(ones_v, acc.at[dst2.at[0]], sem).wait()
    pltpu.make_async_copy(ones_v, acc.at[dst2.at[1]], sem).wait()
    plsc.subcore_barrier()
    _sc_writeout(acc, out_hbm, s, ob, valid)
    plsc.subcore_barrier()


@functools.lru_cache(maxsize=None)
def _sc_counts_pass_k():
    mesh = plsc.VectorSubcoreMesh(core_axis_name="c", subcore_axis_name="s")

    @functools.partial(
        pl.kernel, mesh=mesh,
        out_type=jax.ShapeDtypeStruct((70000, 128), jnp.float32),
        scratch_types=[
            pltpu.VMEM((2, CHUNK), jnp.int32),
            pltpu.VMEM((CHUNK, 128), jnp.float32),
            pltpu.VMEM_SHARED((ACC_ROWS, 128), jnp.float32),
            pltpu.SemaphoreType.DMA,
        ])
    def _sc_counts_pass(dst_p, dst_r, dst_c0, dst_c1, dst_c2, dst_c3,
                        zerosc_hbm, ones_hbm,
                        out_hbm, dst2, ones_v, acc, sem):
        c = lax.axis_index("c")
        s = lax.axis_index("s")
        pltpu.sync_copy(ones_hbm, ones_v)
        core_slots = (
            ((dst_p, NC0, 0, 10000),
             (dst_c0, NC2, 20000, CW[0]),
             (dst_c1, NC2, 20000 + CB[1], CW[1])),
            ((dst_r, NC0, 10000, 10000),
             (dst_c2, NC2, 20000 + CB[2], CW[2]),
             (dst_c3, NC2, 20000 + CB[3], CW[3])),
        )
        for core_id in (0, 1):
            @pl.when(c == core_id)
            def _():
                for (dstR, nchunk, ob, valid) in core_slots[core_id]:
                    _run_cslot(dstR, nchunk, ob, valid, out_hbm, zerosc_hbm,
                               dst2, ones_v, acc, sem, s)

    return _sc_counts_pass


BR = 2000


def _proj_body(x_ref, w_ref, b_ref, o_ref):
    o_ref[...] = jnp.dot(x_ref[...], w_ref[...],
                         preferred_element_type=jnp.float32) + b_ref[...]


def _tc_proj(x, w, b):
    n = x.shape[0]
    return pl.pallas_call(
        _proj_body,
        grid=(n // BR,),
        in_specs=[pl.BlockSpec((BR, 128), lambda i: (i, 0)),
                  pl.BlockSpec((128, 128), lambda i: (0, 0)),
                  pl.BlockSpec((1, 128), lambda i: (0, 0))],
        out_specs=pl.BlockSpec((BR, 128), lambda i: (i, 0)),
        out_shape=jax.ShapeDtypeStruct((n, 128), jnp.float32),
    )(x, w, b.reshape(1, 128))


_BN_SCALE = 1.0 / (1.0 + 1e-5) ** 0.5


def _mean(s_ref, c_ref):
    inv = 1.0 / jnp.maximum(c_ref[...][:, 0:1], 1.0)
    return s_ref[...] * inv


def _combine1_body(x_ref, wroot_ref, s_ref, c_ref, w_ref, b_ref, g_ref, be_ref,
                   o_ref, *, relu):
    acc = jnp.dot(x_ref[...], wroot_ref[...], preferred_element_type=jnp.float32)
    acc += jnp.dot(_mean(s_ref, c_ref), w_ref[...], preferred_element_type=jnp.float32)
    acc += b_ref[...]
    acc = g_ref[...] * acc * _BN_SCALE + be_ref[...]
    o_ref[...] = jnp.maximum(acc, 0.0) if relu else acc


def _combine2_body(x_ref, wroot_ref, s1_ref, c1_ref, w1_ref, s2_ref, c2_ref, w2_ref,
                   b_ref, g_ref, be_ref, o_ref, *, relu):
    acc = jnp.dot(x_ref[...], wroot_ref[...], preferred_element_type=jnp.float32)
    acc += jnp.dot(_mean(s1_ref, c1_ref), w1_ref[...], preferred_element_type=jnp.float32)
    acc += jnp.dot(_mean(s2_ref, c2_ref), w2_ref[...], preferred_element_type=jnp.float32)
    acc += b_ref[...]
    acc = g_ref[...] * acc * _BN_SCALE + be_ref[...]
    o_ref[...] = jnp.maximum(acc, 0.0) if relu else acc


def _row_spec(off, w):
    o = off // BR
    return pl.BlockSpec((BR, w), lambda i, o=o: (i + o, 0))


def _full_spec(r, cdim):
    return pl.BlockSpec((r, cdim), lambda i: (0, 0))


def _combine1(nrows, x, xoff, S, soff, cnt, wroot, w, b, g, be, relu):
    dout = w.shape[1]
    body = functools.partial(_combine1_body, relu=relu)
    return pl.pallas_call(
        body,
        grid=(nrows // BR,),
        in_specs=[_row_spec(xoff, 128), _full_spec(128, dout),
                  _row_spec(soff, 128), _row_spec(soff, 128), _full_spec(128, dout),
                  _full_spec(1, dout), _full_spec(1, dout), _full_spec(1, dout)],
        out_specs=pl.BlockSpec((BR, dout), lambda i: (i, 0)),
        out_shape=jax.ShapeDtypeStruct((nrows, dout), jnp.float32),
    )(x, wroot, S, cnt, w, b.reshape(1, dout), g.reshape(1, dout), be.reshape(1, dout))


def _combine2(nrows, x, xoff, S, s1off, s2off, cnt, wroot, w1, w2, b, g, be, relu):
    dout = w1.shape[1]
    body = functools.partial(_combine2_body, relu=relu)
    return pl.pallas_call(
        body,
        grid=(nrows // BR,),
        in_specs=[_row_spec(xoff, 128), _full_spec(128, dout),
                  _row_spec(s1off, 128), _row_spec(s1off, 128), _full_spec(128, dout),
                  _row_spec(s2off, 128), _row_spec(s2off, 128), _full_spec(128, dout),
                  _full_spec(1, dout), _full_spec(1, dout), _full_spec(1, dout)],
        out_specs=pl.BlockSpec((BR, dout), lambda i: (i, 0)),
        out_shape=jax.ShapeDtypeStruct((nrows, dout), jnp.float32),
    )(x, wroot, S, cnt, w1, S, cnt, w2,
      b.reshape(1, dout), g.reshape(1, dout), be.reshape(1, dout))


def _head2_body(x_ref, wroot_ref, s1_ref, c1_ref, w1_ref, s2_ref, c2_ref, w2_ref,
                b_ref, g_ref, be_ref, wc1_ref, bc1_ref, wc2_ref, bc2_ref, o_ref,
                *, two_rel):
    acc = jnp.dot(x_ref[...], wroot_ref[...], preferred_element_type=jnp.float32)
    acc += jnp.dot(_mean(s1_ref, c1_ref), w1_ref[...], preferred_element_type=jnp.float32)
    if two_rel:
        acc += jnp.dot(_mean(s2_ref, c2_ref), w2_ref[...], preferred_element_type=jnp.float32)
    acc += b_ref[...]
    acc = g_ref[...] * acc * _BN_SCALE + be_ref[...]
    h = jnp.maximum(jnp.dot(acc, wc1_ref[...], preferred_element_type=jnp.float32)
                    + bc1_ref[...], 0.0)
    o_ref[...] = jnp.dot(h, wc2_ref[...], preferred_element_type=jnp.float32) + bc2_ref[...]


def _head(nrows, x, xoff, S, s1off, s2off, cnt, wroot, w1, w2, b, g, be,
          wc1, bc1, wc2p, bc2p, two_rel):
    body = functools.partial(_head2_body, two_rel=two_rel)
    return pl.pallas_call(
        body,
        grid=(nrows // BR,),
        in_specs=[_row_spec(xoff, 128), _full_spec(128, 64),
                  _row_spec(s1off, 128), _row_spec(s1off, 128), _full_spec(128, 64),
                  _row_spec(s2off, 128), _row_spec(s2off, 128), _full_spec(128, 64),
                  _full_spec(1, 64), _full_spec(1, 64), _full_spec(1, 64),
                  _full_spec(64, 64), _full_spec(1, 64),
                  _full_spec(64, 128), _full_spec(1, 128)],
        out_specs=pl.BlockSpec((BR, 128), lambda i: (i, 0)),
        out_shape=jax.ShapeDtypeStruct((nrows, 128), jnp.float32),
    )(x, wroot, S, cnt, w1, S, cnt, w2,
      b.reshape(1, 64), g.reshape(1, 64), be.reshape(1, 64),
      wc1, bc1.reshape(1, 64), wc2p, bc2p.reshape(1, 128))


def _pad1(a, n, val):
    return jnp.concatenate([a.astype(jnp.int32),
                            jnp.full((n - a.shape[0],), val, jnp.int32)])


def _pack(srcA, dstA):
    n = srcA.shape[0] // CHUNK
    return jnp.stack([srcA.reshape(n, CHUNK), dstA.reshape(n, CHUNK)],
                     axis=1).reshape(n * 2, CHUNK)


def _slot_arrays(eip, eir, eit):
    s_p = _pad1(eip[0], E0P, 0)
    d_p = _pad1(eip[1], E0P, GR)
    s_r = _pad1(eir[0] + N_ACC, E0P, 0)
    d_r = _pad1(eir[1], E0P, GR)
    s_t = _pad1(eit[0], E2P, 0)
    dt = _pad1(eit[1], E2P, -1)

    def chunk_dst(k):
        inr = (dt >= CB[k]) & (dt < CB[k] + CW[k])
        return jnp.where(inr, dt - CB[k], GR)

    chunks = tuple(chunk_dst(k) for k in range(4))
    packs = (_pack(s_p, d_p), _pack(s_r, d_r)) + tuple(
        _pack(s_t, ch) for ch in chunks)
    return packs, (d_p, d_r) + chunks


def kernel(x_account, x_merchant, edge_index_pays, edge_index_receives,
           edge_index_transfers, Wp_account, bp_account, Wp_merchant, bp_merchant,
           Wrel0, Wroot0, b0, gamma0, beta0,
           Wrel1, Wroot1, b1, gamma1, beta1,
           Wrel2, Wroot2, b2, gamma2, beta2,
           Wc1, bc1, Wc2, bc2):
    packs, dsts = _slot_arrays(
        edge_index_pays, edge_index_receives, edge_index_transfers)
    zeros_hbm = jnp.zeros((ZROWS, 128), jnp.float32)
    ones_hbm = jnp.ones((CHUNK, 128), jnp.float32)

    cnt = _sc_counts_pass_k()(*dsts, zeros_hbm, ones_hbm)
    feature_pass = _sc_feature_pass_k()

    ha = _tc_proj(x_account, Wp_account, bp_account)
    hm = _tc_proj(x_merchant, Wp_merchant, bp_merchant)
    x = jnp.concatenate([ha, hm], axis=0)

    layers = ((Wrel0, Wroot0, b0, gamma0, beta0),
              (Wrel1, Wroot1, b1, gamma1, beta1))
    for (Wrel, Wroot, b, g, be) in layers:
        S = feature_pass(x, *packs, zeros_hbm)
        pA = _combine2(10000, x, 0, S, 10000, 20000, cnt, Wroot,
                       Wrel[1], Wrel[2], b, g, be, True)
        pB = _combine1(40000, x, 10000, S, 30000, cnt, Wroot,
                       Wrel[2], b, g, be, True)
        pC = _combine1(10000, x, 50000, S, 0, cnt, Wroot,
                       Wrel[0], b, g, be, True)
        x = jnp.concatenate([pA, pB, pC], axis=0)

    S = feature_pass(x, *packs, zeros_hbm)
    wc2p = jnp.zeros((64, 128), jnp.float32).at[:, :2].set(Wc2)
    bc2p = jnp.zeros((128,), jnp.float32).at[:2].set(bc2)
    lA = _head(10000, x, 0, S, 10000, 20000, cnt, Wroot2,
               Wrel2[1], Wrel2[2], b2, gamma2, beta2, Wc1, bc1, wc2p, bc2p, True)
    lB = _head(40000, x, 10000, S, 30000, 30000, cnt, Wroot2,
               Wrel2[2], Wrel2[2], b2, gamma2, beta2, Wc1, bc1, wc2p, bc2p, False)
    logits = jnp.concatenate([lA, lB], axis=0)[:, :2]
    return logits


# speedup vs baseline: 8.6706x; 1.0145x over previous
import functools
import jax
import jax.numpy as jnp
from jax import lax
from jax.experimental import pallas as pl
from jax.experimental.pallas import tpu as pltpu
from jax.experimental.pallas import tpu_sc as plsc

N_ACC = 50000
N_MER = 10000
N = N_ACC + N_MER

NTILES = 16
CHUNK = 112
ACC_ROWS = 12544
GR = 12520
ZROWS = ACC_ROWS // NTILES

E0P = 250880
E2P = 100352
NC0 = E0P // NTILES // CHUNK
NC2 = E2P // NTILES // CHUNK

CB = (0, 12504, 25008, 37512)
CW = (12504, 12504, 12504, 12488)

def _sc_writeout(acc, out_hbm, s, ob, valid):
    per = (valid // NTILES) & ~7
    rem = valid - per * NTILES
    pltpu.sync_copy(acc.at[pl.ds(s * per, per)], out_hbm.at[pl.ds(ob + s * per, per)])
    @pl.when(s == 0)
    def _():
        pltpu.sync_copy(acc.at[pl.ds(per * NTILES, rem)],
                        out_hbm.at[pl.ds(ob + per * NTILES, rem)])


def _run_slot(packR, nchunk, ob, valid, x_hbm, out_hbm, zeros_hbm,
              grp, rows2, acc, sem, sem2, s):
    ngrp = nchunk // 4
    pltpu.sync_copy(zeros_hbm, acc.at[pl.ds(s * ZROWS, ZROWS)])
    base_row = s * nchunk * 2
    pltpu.sync_copy(packR.at[pl.ds(base_row, 8)], grp.at[0])
    plsc.subcore_barrier()
    pltpu.async_copy(x_hbm.at[grp.at[0, 0]], rows2.at[0], sem)

    def body(G, carry):
        pG = lax.rem(G, 2)
        qG = 1 - pG
        for k in range(4):
            k1 = k & 1
            pltpu.make_async_copy(x_hbm.at[grp.at[pG, 2 * k]],
                                  rows2.at[k1], sem).wait()
            if k == 0:
                @pl.when(G >= 1)
                def _():
                    pltpu.make_async_copy(rows2.at[1 - k1],
                                          acc.at[grp.at[qG, 7]], sem2).wait()

                @pl.when(G < ngrp - 1)
                def _():
                    pltpu.sync_copy(
                        packR.at[pl.ds(base_row + (G + 1) * 8, 8)], grp.at[qG])
            else:
                pltpu.make_async_copy(rows2.at[1 - k1],
                                      acc.at[grp.at[pG, 2 * k - 1]], sem2).wait()
            if k < 3:
                pltpu.async_copy(x_hbm.at[grp.at[pG, 2 * k + 2]],
                                 rows2.at[1 - k1], sem)
            else:
                @pl.when(G < ngrp - 1)
                def _():
                    pltpu.async_copy(x_hbm.at[grp.at[qG, 0]],
                                     rows2.at[1 - k1], sem)
            pltpu.async_copy(rows2.at[k1], acc.at[grp.at[pG, 2 * k + 1]],
                             sem2, add=True)
        return carry

    lax.fori_loop(0, ngrp, body, 0)
    pLast = (ngrp - 1) % 2
    pltpu.make_async_copy(rows2.at[1], acc.at[grp.at[pLast, 7]], sem2).wait()
    plsc.subcore_barrier()
    _sc_writeout(acc, out_hbm, s, ob, valid)
    plsc.subcore_barrier()


@functools.lru_cache(maxsize=None)
def _sc_feature_pass_k():
    mesh = plsc.VectorSubcoreMesh(core_axis_name="c", subcore_axis_name="s")

    @functools.partial(
        pl.kernel, mesh=mesh,
        out_type=jax.ShapeDtypeStruct((70000, 128), jnp.float32),
        scratch_types=[
            pltpu.VMEM((2, 8, CHUNK), jnp.int32),
            pltpu.VMEM((2, CHUNK, 128), jnp.float32),
            pltpu.VMEM_SHARED((ACC_ROWS, 128), jnp.float32),
            pltpu.SemaphoreType.DMA,
            pltpu.SemaphoreType.DMA,
        ])
    def _sc_feature_pass(x_hbm, pk_p, pk_r, pk_c0, pk_c1, pk_c2, pk_c3,
                         zeros_hbm, out_hbm, grp, rows2, acc, sem, sem2):
        c = lax.axis_index("c")
        s = lax.axis_index("s")
        core_slots = (
            ((pk_p, NC0, 0, 10000),
             (pk_c0, NC2, 20000, CW[0]),
             (pk_c1, NC2, 20000 + CB[1], CW[1])),
            ((pk_r, NC0, 10000, 10000),
             (pk_c2, NC2, 20000 + CB[2], CW[2]),
             (pk_c3, NC2, 20000 + CB[3], CW[3])),
        )
        for core_id in (0, 1):
            @pl.when(c == core_id)
            def _():
                for (packR, nchunk, ob, valid) in core_slots[core_id]:
                    _run_slot(packR, nchunk, ob, valid, x_hbm, out_hbm,
                              zeros_hbm, grp, rows2, acc, sem, sem2, s)

    return _sc_feature_pass


def _run_cslot(packR, nchunk, ob, valid, out_hbm, zerosc_hbm,
               grp, ones_v, acc, sem, s):
    ngrp = nchunk // 4
    pltpu.sync_copy(zerosc_hbm, acc.at[pl.ds(s * ZROWS, ZROWS)])
    base_row = s * nchunk * 2
    pltpu.sync_copy(packR.at[pl.ds(base_row, 8)], grp.at[0])
    plsc.subcore_barrier()

    def body(G, carry):
        pG = lax.rem(G, 2)
        qG = 1 - pG

        @pl.when(G >= 1)
        def _():
            for _k in range(4):
                pltpu.make_async_copy(ones_v, acc.at[grp.at[qG, 1]], sem).wait()

        @pl.when(G < ngrp - 1)
        def _():
            pltpu.sync_copy(packR.at[pl.ds(base_row + (G + 1) * 8, 8)],
                            grp.at[qG])

        for k in range(4):
            pltpu.async_copy(ones_v, acc.at[grp.at[pG, 2 * k + 1]],
                             sem, add=True)
        return carry

    lax.fori_loop(0, ngrp, body, 0)
    pLast = (ngrp - 1) % 2
    for _k in range(4):
        pltpu.make_async_copy(ones_v, acc.at[grp.at[pLast, 1]], sem).wait()
    plsc.subcore_barrier()
    _sc_writeout(acc, out_hbm, s, ob, valid)
    plsc.subcore_barrier()


@functools.lru_cache(maxsize=None)
def _sc_counts_pass_k():
    mesh = plsc.VectorSubcoreMesh(core_axis_name="c", subcore_axis_name="s")

    @functools.partial(
        pl.kernel, mesh=mesh,
        out_type=jax.ShapeDtypeStruct((70000, 128), jnp.float32),
        scratch_types=[
            pltpu.VMEM((2, 8, CHUNK), jnp.int32),
            pltpu.VMEM((CHUNK, 128), jnp.float32),
            pltpu.VMEM_SHARED((ACC_ROWS, 128), jnp.float32),
            pltpu.SemaphoreType.DMA,
        ])
    def _sc_counts_pass(pk_p, pk_r, pk_c0, pk_c1, pk_c2, pk_c3,
                        zerosc_hbm, ones_hbm,
                        out_hbm, grp, ones_v, acc, sem):
        c = lax.axis_index("c")
        s = lax.axis_index("s")
        pltpu.sync_copy(ones_hbm, ones_v)
        core_slots = (
            ((pk_p, NC0, 0, 10000),
             (pk_c0, NC2, 20000, CW[0]),
             (pk_c1, NC2, 20000 + CB[1], CW[1])),
            ((pk_r, NC0, 10000, 10000),
             (pk_c2, NC2, 20000 + CB[2], CW[2]),
             (pk_c3, NC2, 20000 + CB[3], CW[3])),
        )
        for core_id in (0, 1):
            @pl.when(c == core_id)
            def _():
                for (packR, nchunk, ob, valid) in core_slots[core_id]:
                    _run_cslot(packR, nchunk, ob, valid, out_hbm, zerosc_hbm,
                               grp, ones_v, acc, sem, s)

    return _sc_counts_pass


BR = 2000


def _proj_body(x_ref, w_ref, b_ref, o_ref):
    o_ref[...] = jnp.dot(x_ref[...], w_ref[...],
                         preferred_element_type=jnp.float32) + b_ref[...]


def _tc_proj(x, w, b):
    n = x.shape[0]
    return pl.pallas_call(
        _proj_body,
        grid=(n // BR,),
        in_specs=[pl.BlockSpec((BR, 128), lambda i: (i, 0)),
                  pl.BlockSpec((128, 128), lambda i: (0, 0)),
                  pl.BlockSpec((1, 128), lambda i: (0, 0))],
        out_specs=pl.BlockSpec((BR, 128), lambda i: (i, 0)),
        out_shape=jax.ShapeDtypeStruct((n, 128), jnp.float32),
    )(x, w, b.reshape(1, 128))


_BN_SCALE = 1.0 / (1.0 + 1e-5) ** 0.5


def _mean(s_ref, c_ref):
    inv = 1.0 / jnp.maximum(c_ref[...][:, 0:1], 1.0)
    return s_ref[...] * inv


def _combine1_body(x_ref, wroot_ref, s_ref, c_ref, w_ref, b_ref, g_ref, be_ref,
                   o_ref, *, relu):
    acc = jnp.dot(x_ref[...], wroot_ref[...], preferred_element_type=jnp.float32)
    acc += jnp.dot(_mean(s_ref, c_ref), w_ref[...], preferred_element_type=jnp.float32)
    acc += b_ref[...]
    acc = g_ref[...] * acc * _BN_SCALE + be_ref[...]
    o_ref[...] = jnp.maximum(acc, 0.0) if relu else acc


def _combine2_body(x_ref, wroot_ref, s1_ref, c1_ref, w1_ref, s2_ref, c2_ref, w2_ref,
                   b_ref, g_ref, be_ref, o_ref, *, relu):
    acc = jnp.dot(x_ref[...], wroot_ref[...], preferred_element_type=jnp.float32)
    acc += jnp.dot(_mean(s1_ref, c1_ref), w1_ref[...], preferred_element_type=jnp.float32)
    acc += jnp.dot(_mean(s2_ref, c2_ref), w2_ref[...], preferred_element_type=jnp.float32)
    acc += b_ref[...]
    acc = g_ref[...] * acc * _BN_SCALE + be_ref[...]
    o_ref[...] = jnp.maximum(acc, 0.0) if relu else acc


def _row_spec(off, w):
    o = off // BR
    return pl.BlockSpec((BR, w), lambda i, o=o: (i + o, 0))


def _full_spec(r, cdim):
    return pl.BlockSpec((r, cdim), lambda i: (0, 0))


def _combine1(nrows, x, xoff, S, soff, cnt, wroot, w, b, g, be, relu):
    dout = w.shape[1]
    body = functools.partial(_combine1_body, relu=relu)
    return pl.pallas_call(
        body,
        grid=(nrows // BR,),
        in_specs=[_row_spec(xoff, 128), _full_spec(128, dout),
                  _row_spec(soff, 128), _row_spec(soff, 128), _full_spec(128, dout),
                  _full_spec(1, dout), _full_spec(1, dout), _full_spec(1, dout)],
        out_specs=pl.BlockSpec((BR, dout), lambda i: (i, 0)),
        out_shape=jax.ShapeDtypeStruct((nrows, dout), jnp.float32),
    )(x, wroot, S, cnt, w, b.reshape(1, dout), g.reshape(1, dout), be.reshape(1, dout))


def _combine2(nrows, x, xoff, S, s1off, s2off, cnt, wroot, w1, w2, b, g, be, relu):
    dout = w1.shape[1]
    body = functools.partial(_combine2_body, relu=relu)
    return pl.pallas_call(
        body,
        grid=(nrows // BR,),
        in_specs=[_row_spec(xoff, 128), _full_spec(128, dout),
                  _row_spec(s1off, 128), _row_spec(s1off, 128), _full_spec(128, dout),
                  _row_spec(s2off, 128), _row_spec(s2off, 128), _full_spec(128, dout),
                  _full_spec(1, dout), _full_spec(1, dout), _full_spec(1, dout)],
        out_specs=pl.BlockSpec((BR, dout), lambda i: (i, 0)),
        out_shape=jax.ShapeDtypeStruct((nrows, dout), jnp.float32),
    )(x, wroot, S, cnt, w1, S, cnt, w2,
      b.reshape(1, dout), g.reshape(1, dout), be.reshape(1, dout))


def _head2_body(x_ref, wroot_ref, s1_ref, c1_ref, w1_ref, s2_ref, c2_ref, w2_ref,
                b_ref, g_ref, be_ref, wc1_ref, bc1_ref, wc2_ref, bc2_ref, o_ref,
                *, two_rel):
    acc = jnp.dot(x_ref[...], wroot_ref[...], preferred_element_type=jnp.float32)
    acc += jnp.dot(_mean(s1_ref, c1_ref), w1_ref[...], preferred_element_type=jnp.float32)
    if two_rel:
        acc += jnp.dot(_mean(s2_ref, c2_ref), w2_ref[...], preferred_element_type=jnp.float32)
    acc += b_ref[...]
    acc = g_ref[...] * acc * _BN_SCALE + be_ref[...]
    h = jnp.maximum(jnp.dot(acc, wc1_ref[...], preferred_element_type=jnp.float32)
                    + bc1_ref[...], 0.0)
    o_ref[...] = jnp.dot(h, wc2_ref[...], preferred_element_type=jnp.float32) + bc2_ref[...]


def _head(nrows, x, xoff, S, s1off, s2off, cnt, wroot, w1, w2, b, g, be,
          wc1, bc1, wc2p, bc2p, two_rel):
    body = functools.partial(_head2_body, two_rel=two_rel)
    return pl.pallas_call(
        body,
        grid=(nrows // BR,),
        in_specs=[_row_spec(xoff, 128), _full_spec(128, 64),
                  _row_spec(s1off, 128), _row_spec(s1off, 128), _full_spec(128, 64),
                  _row_spec(s2off, 128), _row_spec(s2off, 128), _full_spec(128, 64),
                  _full_spec(1, 64), _full_spec(1, 64), _full_spec(1, 64),
                  _full_spec(64, 64), _full_spec(1, 64),
                  _full_spec(64, 128), _full_spec(1, 128)],
        out_specs=pl.BlockSpec((BR, 128), lambda i: (i, 0)),
        out_shape=jax.ShapeDtypeStruct((nrows, 128), jnp.float32),
    )(x, wroot, S, cnt, w1, S, cnt, w2,
      b.reshape(1, 64), g.reshape(1, 64), be.reshape(1, 64),
      wc1, bc1.reshape(1, 64), wc2p, bc2p.reshape(1, 128))


def _pad1(a, n, val):
    return jnp.concatenate([a.astype(jnp.int32),
                            jnp.full((n - a.shape[0],), val, jnp.int32)])


def _pack(srcA, dstA):
    n = srcA.shape[0] // CHUNK
    return jnp.stack([srcA.reshape(n, CHUNK), dstA.reshape(n, CHUNK)],
                     axis=1).reshape(n * 2, CHUNK)


def _slot_arrays(eip, eir, eit):
    s_p = _pad1(eip[0], E0P, 0)
    d_p = _pad1(eip[1], E0P, GR)
    s_r = _pad1(eir[0] + N_ACC, E0P, 0)
    d_r = _pad1(eir[1], E0P, GR)
    s_t = _pad1(eit[0], E2P, 0)
    dt = _pad1(eit[1], E2P, -1)

    def chunk_dst(k):
        inr = (dt >= CB[k]) & (dt < CB[k] + CW[k])
        return jnp.where(inr, dt - CB[k], GR)

    chunks = tuple(chunk_dst(k) for k in range(4))
    packs = (_pack(s_p, d_p), _pack(s_r, d_r)) + tuple(
        _pack(s_t, ch) for ch in chunks)
    return packs


def kernel(x_account, x_merchant, edge_index_pays, edge_index_receives,
           edge_index_transfers, Wp_account, bp_account, Wp_merchant, bp_merchant,
           Wrel0, Wroot0, b0, gamma0, beta0,
           Wrel1, Wroot1, b1, gamma1, beta1,
           Wrel2, Wroot2, b2, gamma2, beta2,
           Wc1, bc1, Wc2, bc2):
    packs = _slot_arrays(
        edge_index_pays, edge_index_receives, edge_index_transfers)
    zeros_hbm = jnp.zeros((ZROWS, 128), jnp.float32)
    ones_hbm = jnp.ones((CHUNK, 128), jnp.float32)

    cnt = _sc_counts_pass_k()(*packs, zeros_hbm, ones_hbm)
    feature_pass = _sc_feature_pass_k()

    ha = _tc_proj(x_account, Wp_account, bp_account)
    hm = _tc_proj(x_merchant, Wp_merchant, bp_merchant)
    x = jnp.concatenate([ha, hm], axis=0)

    layers = ((Wrel0, Wroot0, b0, gamma0, beta0),
              (Wrel1, Wroot1, b1, gamma1, beta1))
    for (Wrel, Wroot, b, g, be) in layers:
        S = feature_pass(x, *packs, zeros_hbm)
        pA = _combine2(10000, x, 0, S, 10000, 20000, cnt, Wroot,
                       Wrel[1], Wrel[2], b, g, be, True)
        pB = _combine1(40000, x, 10000, S, 30000, cnt, Wroot,
                       Wrel[2], b, g, be, True)
        pC = _combine1(10000, x, 50000, S, 0, cnt, Wroot,
                       Wrel[0], b, g, be, True)
        x = jnp.concatenate([pA, pB, pC], axis=0)

    S = feature_pass(x, *packs, zeros_hbm)
    wc2p = jnp.zeros((64, 128), jnp.float32).at[:, :2].set(Wc2)
    bc2p = jnp.zeros((128,), jnp.float32).at[:2].set(bc2)
    lA = _head(10000, x, 0, S, 10000, 20000, cnt, Wroot2,
               Wrel2[1], Wrel2[2], b2, gamma2, beta2, Wc1, bc1, wc2p, bc2p, True)
    lB = _head(40000, x, 10000, S, 30000, 30000, cnt, Wroot2,
               Wrel2[2], Wrel2[2], b2, gamma2, beta2, Wc1, bc1, wc2p, bc2p, False)
    logits = jnp.concatenate([lA, lB], axis=0)[:, :2]
    return logits
